# baseline (speedup 1.0000x reference)
import functools

import jax
import jax.numpy as jnp
from jax import lax
from jax.experimental import pallas as pl
from jax.experimental.pallas import tpu as pltpu
from jax.experimental.pallas import tpu_sc as plsc

N = 10000
E = 320000
D_IN = 128
H = 96
C = 64
BN_EPS = 1e-5

NC = 2
NS = 16
NW = NC * NS
EPW = E // NW
LANE_WIN = 128
NWIN = EPW // LANE_WIN
TAIL = EPW - NWIN * LANE_WIN
NPAD = 12000
ZROWS = NPAD // NS

ROW_BLK = 2000


@functools.partial(
    pl.kernel,
    out_type=jax.ShapeDtypeStruct((2 * NPAD, H), jnp.bfloat16),
    mesh=plsc.VectorSubcoreMesh(core_axis_name="c", subcore_axis_name="s"),
    compiler_params=pltpu.CompilerParams(use_tc_tiling_on_sc=False),
    scratch_types=[
        pltpu.VMEM((EPW,), jnp.int32),
        pltpu.VMEM((EPW,), jnp.int32),
        [pltpu.VMEM((LANE_WIN, H), jnp.bfloat16) for _ in range(6)],
        pltpu.VMEM((TAIL, H), jnp.bfloat16),
        pltpu.VMEM_SHARED((NPAD, H), jnp.bfloat16),
        [pltpu.SemaphoreType.DMA for _ in range(6)],
        [pltpu.SemaphoreType.DMA for _ in range(6)],
        pltpu.SemaphoreType.DMA,
    ],
)
def _sc_agg(edge_hbm, x_hbm, zero_hbm, out_hbm,
            src_v, dst_v, rows, rows_t, agg_sh, gsem, ssem, stsem):
    c = lax.axis_index("c")
    s = lax.axis_index("s")
    wid = s * NC + c

    def fire_gather(w, b):
        pltpu.async_copy(x_hbm.at[src_v.at[pl.ds(w * LANE_WIN, LANE_WIN)]],
                         rows[b], gsem[b])

    def wait_gather(b):
        pltpu.make_async_copy(x_hbm.at[src_v.at[pl.ds(0, LANE_WIN)]],
                              rows[b], gsem[b]).wait()

    def fire_scatter(w, b):
        pltpu.async_copy(rows[b],
                         agg_sh.at[dst_v.at[pl.ds(w * LANE_WIN, LANE_WIN)]],
                         ssem[b], add=True)

    def wait_scatter(b):
        pltpu.make_async_copy(rows[b],
                              agg_sh.at[dst_v.at[pl.ds(0, LANE_WIN)]],
                              ssem[b]).wait()

    base = wid * EPW
    zcp = pltpu.make_async_copy(zero_hbm, agg_sh.at[pl.ds(s * ZROWS, ZROWS)],
                                stsem)
    zcp.start()
    pltpu.async_copy(edge_hbm.at[0, pl.ds(base, EPW)], src_v, gsem[0])
    pltpu.async_copy(edge_hbm.at[1, pl.ds(base, EPW)], dst_v, gsem[1])
    pltpu.make_async_copy(edge_hbm.at[0, pl.ds(base, EPW)], src_v,
                          gsem[0]).wait()
    pltpu.make_async_copy(edge_hbm.at[1, pl.ds(base, EPW)], dst_v,
                          gsem[1]).wait()
    fire_gather(0, 0)
    fire_gather(1, 1)
    fire_gather(2, 2)
    zcp.wait()
    plsc.subcore_barrier()
    for w in range(3):
        wait_gather(w)
        fire_scatter(w, w)
        fire_gather(w + 3, w + 3)

    def body(i, carry):
        for b6 in range(6):
            w = 3 + 6 * i + b6
            b = (3 + b6) % 6
            wait_gather(b)
            fire_scatter(w, b)
            wait_scatter((b + 3) % 6)
            fire_gather(w + 3, (b + 3) % 6)
        return carry

    lax.fori_loop(0, (NWIN - 6) // 6, body, 0)

    for w in range(NWIN - 3, NWIN):
        b = w % 6
        wait_gather(b)
        fire_scatter(w, b)
        wait_scatter((b + 3) % 6)
    for w in range(NWIN - 3, NWIN):
        wait_scatter(w % 6)

    tail_off = NWIN * LANE_WIN
    pltpu.async_copy(x_hbm.at[src_v.at[pl.ds(tail_off, TAIL)]],
                     rows_t, gsem[0])
    pltpu.make_async_copy(x_hbm.at[src_v.at[pl.ds(tail_off, TAIL)]],
                          rows_t, gsem[0]).wait()
    pltpu.sync_copy(rows_t, agg_sh.at[dst_v.at[pl.ds(tail_off, TAIL)]],
                    add=True)

    plsc.subcore_barrier()

    pltpu.sync_copy(agg_sh.at[pl.ds(s * ZROWS, ZROWS)],
                    out_hbm.at[pl.ds(c * NPAD + s * ZROWS, ZROWS)])


def _embed_body(h_ref, w_ref, b_ref, o_ref):
    o_ref[...] = (
        jnp.dot(h_ref[...], w_ref[...], preferred_element_type=jnp.float32)
        + b_ref[...]
    )


def _embed(h, W_embed, b_embed):
    return pl.pallas_call(
        _embed_body,
        grid=(N // ROW_BLK,),
        in_specs=[
            pl.BlockSpec((ROW_BLK, D_IN), lambda i: (i, 0)),
            pl.BlockSpec((D_IN, H), lambda i: (0, 0)),
            pl.BlockSpec((1, H), lambda i: (0, 0)),
        ],
        out_specs=pl.BlockSpec((ROW_BLK, H), lambda i: (i, 0)),
        out_shape=jax.ShapeDtypeStruct((N, H), jnp.float32),
    )(h, W_embed, b_embed.reshape(1, H))


def _layer_body(x_ref, p0_ref, p1_ref, w_ref, b_ref, g_ref, bt_ref, o_ref):
    y = (x_ref[...]
         + p0_ref[...].astype(jnp.float32)
         + p1_ref[...].astype(jnp.float32))
    z = jnp.dot(y, w_ref[...], preferred_element_type=jnp.float32) + b_ref[...]
    scale = g_ref[...] * jax.lax.rsqrt(jnp.float32(1.0 + BN_EPS))
    o_ref[...] = jnp.maximum(z * scale + bt_ref[...], 0.0)


def _layer(x, p, W, b, gamma, beta):
    return pl.pallas_call(
        _layer_body,
        grid=(N // ROW_BLK,),
        in_specs=[
            pl.BlockSpec((ROW_BLK, H), lambda i: (i, 0)),
            pl.BlockSpec((ROW_BLK, H), lambda i: (i, 0)),
            pl.BlockSpec((ROW_BLK, H), lambda i: (i + NPAD // ROW_BLK, 0)),
            pl.BlockSpec((H, H), lambda i: (0, 0)),
            pl.BlockSpec((1, H), lambda i: (0, 0)),
            pl.BlockSpec((1, H), lambda i: (0, 0)),
            pl.BlockSpec((1, H), lambda i: (0, 0)),
        ],
        out_specs=pl.BlockSpec((ROW_BLK, H), lambda i: (i, 0)),
        out_shape=jax.ShapeDtypeStruct((N, H), jnp.float32),
    )(x, p, p, W, b.reshape(1, H), gamma.reshape(1, H), beta.reshape(1, H))


def _final_body(x_ref, p0_ref, p1_ref, w_ref, b_ref, g_ref, bt_ref,
                wr_ref, br_ref, o_ref):
    y = (x_ref[...]
         + p0_ref[...].astype(jnp.float32)
         + p1_ref[...].astype(jnp.float32))
    z = jnp.dot(y, w_ref[...], preferred_element_type=jnp.float32) + b_ref[...]
    scale = g_ref[...] * jax.lax.rsqrt(jnp.float32(1.0 + BN_EPS))
    a = jnp.maximum(z * scale + bt_ref[...], 0.0)
    logits = (
        jnp.dot(a, wr_ref[...], preferred_element_type=jnp.float32)
        + br_ref[...]
    )
    m = jnp.max(logits, axis=1, keepdims=True)
    sh = logits - m
    lse = jnp.log(jnp.sum(jnp.exp(sh), axis=1, keepdims=True))
    o_ref[...] = sh - lse


def _final(x, p, W, b, gamma, beta, W_read, b_read):
    return pl.pallas_call(
        _final_body,
        grid=(N // ROW_BLK,),
        in_specs=[
            pl.BlockSpec((ROW_BLK, H), lambda i: (i, 0)),
            pl.BlockSpec((ROW_BLK, H), lambda i: (i, 0)),
            pl.BlockSpec((ROW_BLK, H), lambda i: (i + NPAD // ROW_BLK, 0)),
            pl.BlockSpec((H, H), lambda i: (0, 0)),
            pl.BlockSpec((1, H), lambda i: (0, 0)),
            pl.BlockSpec((1, H), lambda i: (0, 0)),
            pl.BlockSpec((1, H), lambda i: (0, 0)),
            pl.BlockSpec((H, C), lambda i: (0, 0)),
            pl.BlockSpec((1, C), lambda i: (0, 0)),
        ],
        out_specs=pl.BlockSpec((ROW_BLK, C), lambda i: (i, 0)),
        out_shape=jax.ShapeDtypeStruct((N, C), jnp.float32),
    )(x, p, p, W, b.reshape(1, H), gamma.reshape(1, H), beta.reshape(1, H),
      W_read, b_read.reshape(1, C))


def kernel(h, edge_index, W_embed, b_embed, W0, b0, gamma0, beta0,
           W1, b1, gamma1, beta1, W_read, b_read):
    zeros_blk = jnp.zeros((ZROWS, H), jnp.bfloat16)

    x = _embed(h, W_embed, b_embed)
    p = _sc_agg(edge_index, x.astype(jnp.bfloat16), zeros_blk)
    x = _layer(x, p, W0, b0, gamma0, beta0)
    p = _sc_agg(edge_index, x.astype(jnp.bfloat16), zeros_blk)
    return _final(x, p, W1, b1, gamma1, beta1, W_read, b_read)


# speedup vs baseline: 1.1034x; 1.1034x over previous
import functools

import jax
import jax.numpy as jnp
from jax import lax
from jax.experimental import pallas as pl
from jax.experimental.pallas import tpu as pltpu
from jax.experimental.pallas import tpu_sc as plsc

N = 10000
E = 320000
D_IN = 128
H = 96
C = 64
BN_EPS = 1e-5

NC = 2
NS = 16
NW = NC * NS
EPW = E // NW
LANE_WIN = 128
NWIN = EPW // LANE_WIN
TAIL = EPW - NWIN * LANE_WIN
NPAD = 10240
ZROWS = NPAD // NS

ROW_BLK = 2000


@functools.partial(
    pl.kernel,
    out_type=(
        jax.ShapeDtypeStruct((NPAD, H), jnp.bfloat16),
        jax.ShapeDtypeStruct((NPAD, H), jnp.bfloat16),
    ),
    mesh=plsc.VectorSubcoreMesh(core_axis_name="c", subcore_axis_name="s"),
    compiler_params=pltpu.CompilerParams(use_tc_tiling_on_sc=False),
    scratch_types=[
        pltpu.VMEM((EPW,), jnp.int32),
        pltpu.VMEM((EPW,), jnp.int32),
        [pltpu.VMEM((LANE_WIN, H), jnp.bfloat16) for _ in range(8)],
        pltpu.VMEM((TAIL, H), jnp.bfloat16),
        pltpu.VMEM_SHARED((NPAD, H), jnp.bfloat16),
        [pltpu.SemaphoreType.DMA for _ in range(8)],
        [pltpu.SemaphoreType.DMA for _ in range(8)],
        pltpu.SemaphoreType.DMA,
    ],
)
def _sc_agg(edge_hbm, x_hbm, zero_hbm, out0_hbm, out1_hbm,
            src_v, dst_v, rows, rows_t, agg_sh, gsem, ssem, stsem):
    c = lax.axis_index("c")
    s = lax.axis_index("s")
    wid = s * NC + c

    def fire_gather(w, b):
        pltpu.async_copy(x_hbm.at[src_v.at[pl.ds(w * LANE_WIN, LANE_WIN)]],
                         rows[b], gsem[b])

    def wait_gather(b):
        pltpu.make_async_copy(x_hbm.at[src_v.at[pl.ds(0, LANE_WIN)]],
                              rows[b], gsem[b]).wait()

    def fire_scatter(w, b):
        pltpu.async_copy(rows[b],
                         agg_sh.at[dst_v.at[pl.ds(w * LANE_WIN, LANE_WIN)]],
                         ssem[b], add=True)

    def wait_scatter(b):
        pltpu.make_async_copy(rows[b],
                              agg_sh.at[dst_v.at[pl.ds(0, LANE_WIN)]],
                              ssem[b]).wait()

    base = wid * EPW
    zcp = pltpu.make_async_copy(zero_hbm, agg_sh.at[pl.ds(s * ZROWS, ZROWS)],
                                stsem)
    zcp.start()
    pltpu.async_copy(edge_hbm.at[0, pl.ds(base, EPW)], src_v, gsem[0])
    pltpu.async_copy(edge_hbm.at[1, pl.ds(base, EPW)], dst_v, gsem[1])
    pltpu.make_async_copy(edge_hbm.at[0, pl.ds(base, EPW)], src_v,
                          gsem[0]).wait()
    pltpu.make_async_copy(edge_hbm.at[1, pl.ds(base, EPW)], dst_v,
                          gsem[1]).wait()
    for b in range(4):
        fire_gather(b, b)
    zcp.wait()
    plsc.subcore_barrier()
    for w in range(4):
        wait_gather(w)
        fire_scatter(w, w)
        fire_gather(w + 4, w + 4)

    def body(i, carry):
        for b8 in range(8):
            w = 4 + 8 * i + b8
            b = (4 + b8) % 8
            wait_gather(b)
            fire_scatter(w, b)
            wait_scatter((b + 4) % 8)
            fire_gather(w + 4, (b + 4) % 8)
        return carry

    LOOP_I = (NWIN - 14) // 8
    lax.fori_loop(0, LOOP_I, body, 0)

    for w in range(4 + 8 * LOOP_I, NWIN):
        b = w % 8
        wait_gather(b)
        fire_scatter(w, b)
        wait_scatter((b + 4) % 8)
        if w + 4 < NWIN:
            fire_gather(w + 4, (b + 4) % 8)
    for w in range(NWIN - 4, NWIN):
        wait_scatter(w % 8)

    tail_off = NWIN * LANE_WIN
    pltpu.async_copy(x_hbm.at[src_v.at[pl.ds(tail_off, TAIL)]],
                     rows_t, gsem[0])
    pltpu.make_async_copy(x_hbm.at[src_v.at[pl.ds(tail_off, TAIL)]],
                          rows_t, gsem[0]).wait()
    pltpu.sync_copy(rows_t, agg_sh.at[dst_v.at[pl.ds(tail_off, TAIL)]],
                    add=True)

    plsc.subcore_barrier()

    @pl.when(c == 0)
    def _():
        pltpu.sync_copy(agg_sh.at[pl.ds(s * ZROWS, ZROWS)],
                        out0_hbm.at[pl.ds(s * ZROWS, ZROWS)])

    @pl.when(c == 1)
    def _():
        pltpu.sync_copy(agg_sh.at[pl.ds(s * ZROWS, ZROWS)],
                        out1_hbm.at[pl.ds(s * ZROWS, ZROWS)])


def _embed_body(h_ref, w_ref, b_ref, o_ref):
    o_ref[...] = (
        jnp.dot(h_ref[...], w_ref[...], preferred_element_type=jnp.float32)
        + b_ref[...]
    )


def _embed(h, W_embed, b_embed):
    return pl.pallas_call(
        _embed_body,
        grid=(N // ROW_BLK,),
        in_specs=[
            pl.BlockSpec((ROW_BLK, D_IN), lambda i: (i, 0)),
            pl.BlockSpec((D_IN, H), lambda i: (0, 0)),
            pl.BlockSpec((1, H), lambda i: (0, 0)),
        ],
        out_specs=pl.BlockSpec((ROW_BLK, H), lambda i: (i, 0)),
        out_shape=jax.ShapeDtypeStruct((N, H), jnp.float32),
    )(h, W_embed, b_embed.reshape(1, H))


def _layer_body(x_ref, p0_ref, p1_ref, w_ref, b_ref, g_ref, bt_ref, o_ref):
    y = (x_ref[...]
         + p0_ref[...].astype(jnp.float32)
         + p1_ref[...].astype(jnp.float32))
    z = jnp.dot(y, w_ref[...], preferred_element_type=jnp.float32) + b_ref[...]
    scale = g_ref[...] * jax.lax.rsqrt(jnp.float32(1.0 + BN_EPS))
    o_ref[...] = jnp.maximum(z * scale + bt_ref[...], 0.0)


def _layer(x, p0, p1, W, b, gamma, beta):
    return pl.pallas_call(
        _layer_body,
        grid=(N // ROW_BLK,),
        in_specs=[
            pl.BlockSpec((ROW_BLK, H), lambda i: (i, 0)),
            pl.BlockSpec((ROW_BLK, H), lambda i: (i, 0)),
            pl.BlockSpec((ROW_BLK, H), lambda i: (i, 0)),
            pl.BlockSpec((H, H), lambda i: (0, 0)),
            pl.BlockSpec((1, H), lambda i: (0, 0)),
            pl.BlockSpec((1, H), lambda i: (0, 0)),
            pl.BlockSpec((1, H), lambda i: (0, 0)),
        ],
        out_specs=pl.BlockSpec((ROW_BLK, H), lambda i: (i, 0)),
        out_shape=jax.ShapeDtypeStruct((N, H), jnp.float32),
    )(x, p0, p1, W, b.reshape(1, H), gamma.reshape(1, H), beta.reshape(1, H))


def _final_body(x_ref, p0_ref, p1_ref, w_ref, b_ref, g_ref, bt_ref,
                wr_ref, br_ref, o_ref):
    y = (x_ref[...]
         + p0_ref[...].astype(jnp.float32)
         + p1_ref[...].astype(jnp.float32))
    z = jnp.dot(y, w_ref[...], preferred_element_type=jnp.float32) + b_ref[...]
    scale = g_ref[...] * jax.lax.rsqrt(jnp.float32(1.0 + BN_EPS))
    a = jnp.maximum(z * scale + bt_ref[...], 0.0)
    logits = (
        jnp.dot(a, wr_ref[...], preferred_element_type=jnp.float32)
        + br_ref[...]
    )
    m = jnp.max(logits, axis=1, keepdims=True)
    sh = logits - m
    lse = jnp.log(jnp.sum(jnp.exp(sh), axis=1, keepdims=True))
    o_ref[...] = sh - lse


def _final(x, p0, p1, W, b, gamma, beta, W_read, b_read):
    return pl.pallas_call(
        _final_body,
        grid=(N // ROW_BLK,),
        in_specs=[
            pl.BlockSpec((ROW_BLK, H), lambda i: (i, 0)),
            pl.BlockSpec((ROW_BLK, H), lambda i: (i, 0)),
            pl.BlockSpec((ROW_BLK, H), lambda i: (i, 0)),
            pl.BlockSpec((H, H), lambda i: (0, 0)),
            pl.BlockSpec((1, H), lambda i: (0, 0)),
            pl.BlockSpec((1, H), lambda i: (0, 0)),
            pl.BlockSpec((1, H), lambda i: (0, 0)),
            pl.BlockSpec((H, C), lambda i: (0, 0)),
            pl.BlockSpec((1, C), lambda i: (0, 0)),
        ],
        out_specs=pl.BlockSpec((ROW_BLK, C), lambda i: (i, 0)),
        out_shape=jax.ShapeDtypeStruct((N, C), jnp.float32),
    )(x, p0, p1, W, b.reshape(1, H), gamma.reshape(1, H), beta.reshape(1, H),
      W_read, b_read.reshape(1, C))


def kernel(h, edge_index, W_embed, b_embed, W0, b0, gamma0, beta0,
           W1, b1, gamma1, beta1, W_read, b_read):
    zeros_blk = jnp.zeros((ZROWS, H), jnp.bfloat16)

    x = _embed(h, W_embed, b_embed)
    a0, a1 = _sc_agg(edge_index, x.astype(jnp.bfloat16), zeros_blk)
    x = _layer(x, a0, a1, W0, b0, gamma0, beta0)
    a0, a1 = _sc_agg(edge_index, x.astype(jnp.bfloat16), zeros_blk)
    return _final(x, a0, a1, W1, b1, gamma1, beta1, W_read, b_read)
